# Initial kernel scaffold; baseline (speedup 1.0000x reference)
import jax
import jax.numpy as jnp
from jax.experimental import pallas as pl


def kernel(x, table):
    raise NotImplementedError("write your pallas kernel here")


# speedup vs baseline: 4.2071x; 4.2071x over previous
import jax
import jax.numpy as jnp
from jax.experimental import pallas as pl
from jax.experimental.pallas import tpu as pltpu

SEP = 102
L = 8192
D = 128
ROWS_PER_BLK = 512
NBLK = L // ROWS_PER_BLK


def _body(x_ref, t_ref, o_ref, len_ref):
    pid = pl.program_id(0)

    @pl.when(pid == 0)
    def _():
        xv = x_ref[...]
        r = jax.lax.broadcasted_iota(jnp.int32, (L // D, D), 0)
        c = jax.lax.broadcasted_iota(jnp.int32, (L // D, D), 1)
        gidx = r * D + c
        cand = jnp.where(xv == SEP, gidx, -1)
        last = jnp.max(cand)
        len_ref[0] = jnp.where(last < 0, L, last)

    input_len = len_ref[0]
    rows = pid * ROWS_PER_BLK + jax.lax.broadcasted_iota(
        jnp.int32, (ROWS_PER_BLK, 1), 0)
    mask = rows >= input_len
    o_ref[...] = jnp.where(mask, t_ref[1:2, :], t_ref[0:1, :])


def kernel(x, table):
    x2 = x.reshape(L // D, D)
    return pl.pallas_call(
        _body,
        grid=(NBLK,),
        in_specs=[
            pl.BlockSpec((L // D, D), lambda i: (0, 0)),
            pl.BlockSpec((2, D), lambda i: (0, 0)),
        ],
        out_specs=pl.BlockSpec((ROWS_PER_BLK, D), lambda i: (i, 0)),
        out_shape=jax.ShapeDtypeStruct((L, D), jnp.float32),
        scratch_shapes=[pltpu.SMEM((1,), jnp.int32)],
    )(x2, table)
